# Initial kernel scaffold; baseline (speedup 1.0000x reference)
import jax
import jax.numpy as jnp
from jax.experimental import pallas as pl


def kernel(heat, edge_index, W, b, conv_a, gamma, beta, act_a):
    raise NotImplementedError("write your pallas kernel here")


# speedup vs baseline: 3.0579x; 3.0579x over previous
import functools

import jax
import jax.numpy as jnp
from jax import lax
from jax.experimental import pallas as pl
from jax.experimental.pallas import tpu as pltpu
from jax.experimental.pallas import tpu_sc as plsc

N = 10000
E = 320000
D = 128
EPS = 1e-5

NPAD = 10112
NC = 2
NS = 16
NT = NC * NS
CH = 128
CPT = 80
EPAD = NT * CPT * CH
RPS = NPAD // NS


def _mm_body(x_ref, w_ref, o_ref):
    o_ref[...] = jnp.dot(x_ref[...], w_ref[...],
                         preferred_element_type=jnp.float32)


def _matmul(x, w):
    return pl.pallas_call(
        _mm_body,
        out_shape=jax.ShapeDtypeStruct((NPAD, D), jnp.float32),
    )(x, w)


_SC_MESH = plsc.VectorSubcoreMesh(core_axis_name="c", subcore_axis_name="s")


@functools.partial(
    pl.kernel,
    mesh=_SC_MESH,
    out_type=jax.ShapeDtypeStruct((NC, NPAD, D), jnp.float32),
    scratch_types=[
        pltpu.VMEM((CPT, CH), jnp.int32),
        pltpu.VMEM((CPT, CH), jnp.int32),
        pltpu.VMEM((CH, D), jnp.float32),
        pltpu.VMEM_SHARED((NPAD, D), jnp.float32),
        pltpu.SemaphoreType.DMA,
    ],
)
def _segsum(m_hbm, src_hbm, dst_hbm, zeros_hbm, out_hbm,
            src_v, dst_v, buf_a, acc, sem_a):
    c = lax.axis_index("c")
    s = lax.axis_index("s")
    tid = c * NS + s

    pltpu.sync_copy(src_hbm.at[tid], src_v)
    pltpu.sync_copy(dst_hbm.at[tid], dst_v)
    pltpu.sync_copy(zeros_hbm.at[pl.ds(s * RPS, RPS)],
                    acc.at[pl.ds(s * RPS, RPS)])
    plsc.subcore_barrier()

    def body(j, carry):
        pltpu.async_copy(m_hbm.at[src_v.at[j]], buf_a, sem_a).wait()
        pltpu.sync_copy(buf_a, acc.at[dst_v.at[j]], add=True)
        return carry

    lax.fori_loop(0, CPT, body, 0)

    plsc.subcore_barrier()
    pltpu.sync_copy(acc.at[pl.ds(s * RPS, RPS)],
                    out_hbm.at[c, pl.ds(s * RPS, RPS)])


def _post_body(p_ref, b_ref, g_ref, be_ref, ca_ref, aa_ref, o_ref):
    agg = p_ref[0, 0:N, :] + p_ref[1, 0:N, :] + b_ref[...]
    z = jnp.where(agg > 0, agg, ca_ref[...] * agg)
    mean = jnp.mean(z, axis=0, keepdims=True)
    var = jnp.mean((z - mean) ** 2, axis=0, keepdims=True)
    hn = (z - mean) * lax.rsqrt(var + EPS) * g_ref[...] + be_ref[...]
    o_ref[0:N, :] = jnp.where(hn > 0, hn, aa_ref[...] * hn)
    o_ref[pl.ds(N, NPAD - N), :] = jnp.zeros((NPAD - N, D), jnp.float32)


def _post(parts, b_i, gamma_i, beta_i, ca_i, aa_i):
    return pl.pallas_call(
        _post_body,
        out_shape=jax.ShapeDtypeStruct((NPAD, D), jnp.float32),
    )(parts, b_i, gamma_i, beta_i, ca_i, aa_i)


def kernel(heat, edge_index, W, b, conv_a, gamma, beta, act_a):
    src = edge_index[0]
    dst = edge_index[1]
    pad = EPAD - E
    src_p = jnp.concatenate([src, jnp.full((pad,), N, jnp.int32)])
    dst_p = jnp.concatenate([dst, jnp.full((pad,), N, jnp.int32)])
    src_p = src_p.reshape(NT, CPT, CH)
    dst_p = dst_p.reshape(NT, CPT, CH)
    h = jnp.concatenate(
        [heat, jnp.zeros((NPAD - N, D), jnp.float32)], axis=0)
    zeros = jnp.zeros((NPAD, D), jnp.float32)
    for i in range(2):
        m = _matmul(h, W[i])
        parts = _segsum(m, src_p, dst_p, zeros)
        h = _post(parts,
                  b[i].reshape(1, D),
                  gamma[i].reshape(1, D),
                  beta[i].reshape(1, D),
                  jnp.full((1, D), conv_a[i], jnp.float32),
                  jnp.full((1, D), act_a[i], jnp.float32))
    return h[0:N]


# speedup vs baseline: 3.4709x; 1.1351x over previous
import functools

import jax
import jax.numpy as jnp
from jax import lax
from jax.experimental import pallas as pl
from jax.experimental.pallas import tpu as pltpu
from jax.experimental.pallas import tpu_sc as plsc

N = 10000
E = 320000
D = 128
EPS = 1e-5

NPAD = 10112
NC = 2
NS = 16
NT = NC * NS
CH = 128
CPT = 80
EPAD = NT * CPT * CH
RPS = NPAD // NS


def _mm_body(x_ref, w_ref, o_ref):
    o_ref[...] = jnp.dot(x_ref[...], w_ref[...],
                         preferred_element_type=jnp.float32)


def _matmul(x, w):
    return pl.pallas_call(
        _mm_body,
        out_shape=jax.ShapeDtypeStruct((NPAD, D), jnp.float32),
    )(x, w)


_SC_MESH = plsc.VectorSubcoreMesh(core_axis_name="c", subcore_axis_name="s")


@functools.partial(
    pl.kernel,
    mesh=_SC_MESH,
    out_type=jax.ShapeDtypeStruct((NC, NPAD, D), jnp.float32),
    scratch_types=[
        pltpu.VMEM((CPT // 2, CH), jnp.int32),
        pltpu.VMEM((CPT // 2, CH), jnp.int32),
        pltpu.VMEM((CH, D), jnp.float32),
        pltpu.VMEM((CH, D), jnp.float32),
        pltpu.VMEM_SHARED((NPAD, D), jnp.float32),
        pltpu.SemaphoreType.DMA,
        pltpu.SemaphoreType.DMA,
    ],
)
def _segsum(m_hbm, src_hbm, dst_hbm, zeros_hbm, out_hbm,
            src_v, dst_v, buf_a, buf_b, acc, sem_a, sem_b):
    c = lax.axis_index("c")
    s = lax.axis_index("s")
    tid = c * NS + s
    half = CPT // 2

    pltpu.sync_copy(zeros_hbm.at[pl.ds(s * RPS, RPS)],
                    acc.at[pl.ds(s * RPS, RPS)])
    plsc.subcore_barrier()

    for h in range(2):
        pltpu.sync_copy(src_hbm.at[tid, pl.ds(h * half, half)], src_v)
        pltpu.sync_copy(dst_hbm.at[tid, pl.ds(h * half, half)], dst_v)
        pltpu.async_copy(m_hbm.at[src_v.at[0]], buf_a, sem_a)

        def body(jj, carry):
            j0 = jj * 2
            pltpu.async_copy(m_hbm.at[src_v.at[j0 + 1]], buf_b, sem_b)
            pltpu.make_async_copy(m_hbm.at[src_v.at[j0]], buf_a, sem_a).wait()
            pltpu.sync_copy(buf_a, acc.at[dst_v.at[j0]], add=True)

            @pl.when(jj + 1 < half // 2)
            def _():
                pltpu.async_copy(m_hbm.at[src_v.at[j0 + 2]], buf_a, sem_a)

            pltpu.make_async_copy(m_hbm.at[src_v.at[j0 + 1]], buf_b,
                                  sem_b).wait()
            pltpu.sync_copy(buf_b, acc.at[dst_v.at[j0 + 1]], add=True)
            return carry

        lax.fori_loop(0, half // 2, body, 0)

    plsc.subcore_barrier()
    pltpu.sync_copy(acc.at[pl.ds(s * RPS, RPS)],
                    out_hbm.at[c, pl.ds(s * RPS, RPS)])


def _post_body(p_ref, b_ref, g_ref, be_ref, ca_ref, aa_ref, o_ref):
    agg = p_ref[0, 0:N, :] + p_ref[1, 0:N, :] + b_ref[...]
    z = jnp.where(agg > 0, agg, ca_ref[...] * agg)
    mean = jnp.mean(z, axis=0, keepdims=True)
    var = jnp.mean((z - mean) ** 2, axis=0, keepdims=True)
    hn = (z - mean) * lax.rsqrt(var + EPS) * g_ref[...] + be_ref[...]
    o_ref[0:N, :] = jnp.where(hn > 0, hn, aa_ref[...] * hn)
    o_ref[pl.ds(N, NPAD - N), :] = jnp.zeros((NPAD - N, D), jnp.float32)


def _post(parts, b_i, gamma_i, beta_i, ca_i, aa_i):
    return pl.pallas_call(
        _post_body,
        out_shape=jax.ShapeDtypeStruct((NPAD, D), jnp.float32),
    )(parts, b_i, gamma_i, beta_i, ca_i, aa_i)


def kernel(heat, edge_index, W, b, conv_a, gamma, beta, act_a):
    src = edge_index[0]
    dst = edge_index[1]
    pad = EPAD - E
    src_p = jnp.concatenate([src, jnp.full((pad,), N, jnp.int32)])
    dst_p = jnp.concatenate([dst, jnp.full((pad,), N, jnp.int32)])
    src_p = src_p.reshape(NT, CPT, CH)
    dst_p = dst_p.reshape(NT, CPT, CH)
    h = jnp.concatenate(
        [heat, jnp.zeros((NPAD - N, D), jnp.float32)], axis=0)
    zeros = jnp.zeros((NPAD, D), jnp.float32)
    for i in range(2):
        m = _matmul(h, W[i])
        parts = _segsum(m, src_p, dst_p, zeros)
        h = _post(parts,
                  b[i].reshape(1, D),
                  gamma[i].reshape(1, D),
                  beta[i].reshape(1, D),
                  jnp.full((1, D), conv_a[i], jnp.float32),
                  jnp.full((1, D), act_a[i], jnp.float32))
    return h[0:N]
